# baseline (speedup 1.0000x reference)
import functools

import jax
import jax.numpy as jnp
from jax import lax
from jax.experimental import pallas as pl
from jax.experimental.pallas import tpu as pltpu
from jax.experimental.pallas import tpu_sc as plsc

D = 1024
SEQ = 2048
L = 16
NC = 2
NS = 16
NW = NC * NS
CHUNK = 32
EPS = 1e-12


def _rsqrt16(x):
    i = lax.bitcast_convert_type(x, jnp.int32)
    i = jnp.int32(0x5F3759DF) - lax.shift_right_logical(i, 1)
    r = lax.bitcast_convert_type(i, jnp.float32)
    for _ in range(3):
        r = r * (1.5 - 0.5 * x * r * r)
    return r


def _make_sc_kernel(ntok):
    tpw = ntok // NW
    nchunks = tpw // CHUNK
    mesh = plsc.VectorSubcoreMesh(core_axis_name="c", subcore_axis_name="s")

    @functools.partial(
        pl.kernel,
        out_type=jax.ShapeDtypeStruct((ntok, D), jnp.float32),
        mesh=mesh,
        compiler_params=pltpu.CompilerParams(needs_layout_passes=False),
        scratch_types=[
            pltpu.VMEM((CHUNK,), jnp.int32),
            pltpu.VMEM((CHUNK,), jnp.int32),
            pltpu.VMEM((CHUNK, D), jnp.float32),
            pltpu.VMEM((CHUNK, D), jnp.float32),
            pltpu.VMEM((CHUNK, D), jnp.float32),
            pltpu.VMEM((D,), jnp.float32),
            pltpu.VMEM((D,), jnp.float32),
            pltpu.SemaphoreType.DMA,
        ],
    )
    def sc_embed(ids_hbm, tt_hbm, wtab_hbm, ptab_hbm, ttab_hbm, g_hbm, b_hbm,
                 out_hbm, idx_v, tidx_v, wrow, prow, trow, g_v, b_v, sem):
        wid = lax.axis_index("s") * NC + lax.axis_index("c")
        base = wid * tpw
        pltpu.sync_copy(g_hbm, g_v)
        pltpu.sync_copy(b_hbm, b_v)

        def chunk_body(c, _):
            tok0 = base + c * CHUNK
            s0 = lax.rem(tok0, SEQ)
            pltpu.sync_copy(ids_hbm.at[pl.ds(tok0, CHUNK)], idx_v)
            pltpu.sync_copy(tt_hbm.at[pl.ds(tok0, CHUNK)], tidx_v)
            pltpu.async_copy(wtab_hbm.at[idx_v], wrow, sem).wait()
            pltpu.async_copy(ttab_hbm.at[tidx_v], trow, sem).wait()
            pltpu.sync_copy(ptab_hbm.at[pl.ds(s0, CHUNK)], prow)

            def tok_body(j, _):
                def pass1(k, carry):
                    s, q = carry
                    y = (wrow[j, pl.ds(k * L, L)]
                         + prow[j, pl.ds(k * L, L)]
                         + trow[j, pl.ds(k * L, L)])
                    wrow[j, pl.ds(k * L, L)] = y
                    return s + y, q + y * y

                zero = jnp.zeros((L,), jnp.float32)
                s, q = lax.fori_loop(0, D // L, pass1, (zero, zero))
                inv_d = jnp.float32(1.0 / D)
                mean = jnp.broadcast_to(jnp.sum(s), (L,)) * inv_d
                msq = jnp.broadcast_to(jnp.sum(q), (L,)) * inv_d
                var = msq - mean * mean
                inv = _rsqrt16(var + jnp.float32(EPS))

                def pass2(k, _):
                    y = wrow[j, pl.ds(k * L, L)]
                    a = inv * g_v[pl.ds(k * L, L)]
                    wrow[j, pl.ds(k * L, L)] = (y - mean) * a + b_v[pl.ds(k * L, L)]
                    return 0

                lax.fori_loop(0, D // L, pass2, 0)
                return 0

            lax.fori_loop(0, CHUNK, tok_body, 0)
            pltpu.sync_copy(wrow, out_hbm.at[pl.ds(tok0, CHUNK)])
            return 0

        lax.fori_loop(0, nchunks, chunk_body, 0)

    return sc_embed


def kernel(input_ids, token_type_ids, word_emb, pos_emb, tt_emb, gamma, beta):
    b, seq = input_ids.shape
    ntok = b * seq
    ids_flat = input_ids.reshape(ntok).astype(jnp.int32)
    tt_flat = token_type_ids.reshape(ntok).astype(jnp.int32)
    out = _make_sc_kernel(ntok)(ids_flat, tt_flat, word_emb, pos_emb, tt_emb,
                                gamma, beta)
    return out.reshape(b, seq, D)


# speedup vs baseline: 1.4318x; 1.4318x over previous
import functools

import jax
import jax.numpy as jnp
from jax import lax
from jax.experimental import pallas as pl
from jax.experimental.pallas import tpu as pltpu
from jax.experimental.pallas import tpu_sc as plsc

D = 1024
SEQ = 2048
L = 16
NC = 2
NS = 16
NW = NC * NS
CHUNK = 16
U1 = 4
U2 = 4
EPS = 1e-12


def _rsqrt16(x):
    i = lax.bitcast_convert_type(x, jnp.int32)
    i = jnp.int32(0x5F3759DF) - lax.shift_right_logical(i, 1)
    r = lax.bitcast_convert_type(i, jnp.float32)
    for _ in range(3):
        r = r * (1.5 - 0.5 * x * r * r)
    return r


def _make_sc_kernel(ntok):
    tpw = ntok // NW
    nch = tpw // CHUNK
    mesh = plsc.VectorSubcoreMesh(core_axis_name="c", subcore_axis_name="s")

    @functools.partial(
        pl.kernel,
        out_type=jax.ShapeDtypeStruct((ntok, D), jnp.float32),
        mesh=mesh,
        compiler_params=pltpu.CompilerParams(needs_layout_passes=False),
        scratch_types=[
            pltpu.VMEM((2, CHUNK), jnp.int32),
            pltpu.VMEM((2, CHUNK), jnp.int32),
            pltpu.VMEM((2, CHUNK, D), jnp.float32),
            pltpu.VMEM((2, CHUNK, D), jnp.float32),
            pltpu.VMEM((2, CHUNK, D), jnp.float32),
            pltpu.VMEM((D,), jnp.float32),
            pltpu.VMEM((D,), jnp.float32),
            pltpu.SemaphoreType.DMA((2,)),
            pltpu.SemaphoreType.DMA((2,)),
            pltpu.SemaphoreType.DMA((2,)),
            pltpu.SemaphoreType.DMA((2,)),
        ],
    )
    def sc_embed(ids_hbm, tt_hbm, wtab_hbm, ptab_hbm, ttab_hbm, g_hbm, b_hbm,
                 out_hbm, idxw, idxt, wrow, trow, prow, g_v, b_v,
                 semw, semt, semp, semo):
        wid = lax.axis_index("s") * NC + lax.axis_index("c")
        base = wid * tpw
        pltpu.sync_copy(g_hbm, g_v)
        pltpu.sync_copy(b_hbm, b_v)

        def issue(c, b):
            tok0 = base + c * CHUNK
            s0 = lax.rem(tok0, SEQ)
            pltpu.sync_copy(ids_hbm.at[pl.ds(tok0, CHUNK)], idxw.at[b])
            pltpu.sync_copy(tt_hbm.at[pl.ds(tok0, CHUNK)], idxt.at[b])
            return {
                "w": pltpu.async_copy(wtab_hbm.at[idxw.at[b]], wrow.at[b],
                                      semw.at[b]),
                "t": pltpu.async_copy(ttab_hbm.at[idxt.at[b]], trow.at[b],
                                      semt.at[b]),
                "p": pltpu.async_copy(ptab_hbm.at[pl.ds(s0, CHUNK)],
                                      prow.at[b], semp.at[b]),
            }

        def compute(b):
            wb, tb, pb = wrow.at[b], trow.at[b], prow.at[b]

            def tok_body(j, _):
                def pass1(k2, carry):
                    s, q = carry
                    for u in range(U1):
                        k = k2 * U1 + u
                        y = (wb[j, pl.ds(k * L, L)]
                             + pb[j, pl.ds(k * L, L)]
                             + tb[j, pl.ds(k * L, L)])
                        wb[j, pl.ds(k * L, L)] = y
                        s, q = s + y, q + y * y
                    return s, q

                zero = jnp.zeros((L,), jnp.float32)
                s, q = lax.fori_loop(0, D // L // U1, pass1, (zero, zero))
                inv_d = jnp.float32(1.0 / D)
                mean = jnp.broadcast_to(jnp.sum(s), (L,)) * inv_d
                msq = jnp.broadcast_to(jnp.sum(q), (L,)) * inv_d
                var = msq - mean * mean
                inv = _rsqrt16(var + jnp.float32(EPS))

                def pass2(k2, _):
                    for u in range(U2):
                        k = k2 * U2 + u
                        y = wb[j, pl.ds(k * L, L)]
                        a = inv * g_v[pl.ds(k * L, L)]
                        wb[j, pl.ds(k * L, L)] = ((y - mean) * a
                                                  + b_v[pl.ds(k * L, L)])
                    return 0

                lax.fori_loop(0, D // L // U2, pass2, 0)
                return 0

            lax.fori_loop(0, CHUNK, tok_body, 0)

        desc = {0: issue(0, 0)}
        out_desc = {}
        for c in range(nch):
            b = c & 1
            if c + 1 < nch:
                if c >= 1:
                    out_desc[c - 1].wait()
                desc[c + 1] = issue(c + 1, 1 - b)
            d = desc.pop(c)
            d["w"].wait()
            d["t"].wait()
            d["p"].wait()
            compute(b)
            out_desc[c] = pltpu.async_copy(
                wrow.at[b], out_hbm.at[pl.ds(base + c * CHUNK, CHUNK)],
                semo.at[b])
        out_desc[nch - 2].wait()
        out_desc[nch - 1].wait()

    return sc_embed


def kernel(input_ids, token_type_ids, word_emb, pos_emb, tt_emb, gamma, beta):
    b, seq = input_ids.shape
    ntok = b * seq
    ids_flat = input_ids.reshape(ntok).astype(jnp.int32)
    tt_flat = token_type_ids.reshape(ntok).astype(jnp.int32)
    out = _make_sc_kernel(ntok)(ids_flat, tt_flat, word_emb, pos_emb, tt_emb,
                                gamma, beta)
    return out.reshape(b, seq, D)


# speedup vs baseline: 4.8196x; 3.3662x over previous
import functools

import jax
import jax.numpy as jnp
from jax import lax
from jax.experimental import pallas as pl
from jax.experimental.pallas import tpu as pltpu
from jax.experimental.pallas import tpu_sc as plsc

D = 1024
SEQ = 2048
NC = 2
NS = 16
NW = NC * NS
K = 32
TB = 256
EPS = 1e-12


def _make_sc_gather(ntok):
    tpw = ntok // NW
    nch = tpw // K
    mesh = plsc.VectorSubcoreMesh(core_axis_name="c", subcore_axis_name="s")

    @functools.partial(
        pl.kernel,
        out_type=jax.ShapeDtypeStruct((ntok, D), jnp.float32),
        mesh=mesh,
        compiler_params=pltpu.CompilerParams(needs_layout_passes=False),
        scratch_types=[
            pltpu.VMEM((2, K), jnp.int32),
            pltpu.VMEM((2, K, D), jnp.float32),
            pltpu.SemaphoreType.DMA((2,)),
            pltpu.SemaphoreType.DMA((2,)),
        ],
    )
    def sc_gather(ids_hbm, wtab_hbm, out_hbm, idx, rows, semg, semo):
        wid = lax.axis_index("s") * NC + lax.axis_index("c")
        base = wid * tpw

        def issue(c, b):
            pltpu.sync_copy(ids_hbm.at[pl.ds(base + c * K, K)], idx.at[b])
            return pltpu.async_copy(wtab_hbm.at[idx.at[b]], rows.at[b],
                                    semg.at[b])

        gat = {0: issue(0, 0)}
        out = {}
        for c in range(nch):
            b = c & 1
            if c + 1 < nch:
                if c >= 1:
                    out[c - 1].wait()
                gat[c + 1] = issue(c + 1, 1 - b)
            gat.pop(c).wait()
            out[c] = pltpu.async_copy(
                rows.at[b], out_hbm.at[pl.ds(base + c * K, K)], semo.at[b])
        out[nch - 2].wait()
        out[nch - 1].wait()

    return sc_gather


def _tc_ln_body(wsum_ref, pos_ref, tt_ref, tid_ref, g_ref, b_ref, out_ref):
    tidf = tid_ref[...]
    t0 = tt_ref[0:1, :]
    dt = tt_ref[1:2, :] - t0
    y = wsum_ref[...] + pos_ref[...] + (t0 + tidf * dt)
    mean = jnp.mean(y, axis=-1, keepdims=True)
    var = jnp.mean(y * y, axis=-1, keepdims=True) - mean * mean
    inv = lax.rsqrt(var + EPS)
    out_ref[...] = (y - mean) * inv * g_ref[...] + b_ref[...]


def _make_tc_ln(ntok):
    nblk = ntok // TB
    spb = SEQ // TB
    return pl.pallas_call(
        _tc_ln_body,
        grid=(nblk,),
        in_specs=[
            pl.BlockSpec((TB, D), lambda i: (i, 0)),
            pl.BlockSpec((TB, D), lambda i: (i % spb, 0)),
            pl.BlockSpec((2, D), lambda i: (0, 0)),
            pl.BlockSpec((TB, 1), lambda i: (i, 0)),
            pl.BlockSpec((1, D), lambda i: (0, 0)),
            pl.BlockSpec((1, D), lambda i: (0, 0)),
        ],
        out_specs=pl.BlockSpec((TB, D), lambda i: (i, 0)),
        out_shape=jax.ShapeDtypeStruct((ntok, D), jnp.float32),
    )


def kernel(input_ids, token_type_ids, word_emb, pos_emb, tt_emb, gamma, beta):
    b, seq = input_ids.shape
    ntok = b * seq
    ids_flat = input_ids.reshape(ntok).astype(jnp.int32)
    ttf = token_type_ids.reshape(ntok, 1).astype(jnp.float32)
    wsum = _make_sc_gather(ntok)(ids_flat, word_emb)
    out = _make_tc_ln(ntok)(wsum, pos_emb, tt_emb, ttf,
                            gamma.reshape(1, D), beta.reshape(1, D))
    return out.reshape(b, seq, D)


# speedup vs baseline: 4.9113x; 1.0190x over previous
import functools

import jax
import jax.numpy as jnp
from jax import lax
from jax.experimental import pallas as pl
from jax.experimental.pallas import tpu as pltpu
from jax.experimental.pallas import tpu_sc as plsc

D = 1024
SEQ = 2048
NC = 2
NS = 16
NW = NC * NS
K = 32
TB = 256
EPS = 1e-12


def _make_sc_gather(ntok):
    tpw = ntok // NW
    nch = tpw // K
    mesh = plsc.VectorSubcoreMesh(core_axis_name="c", subcore_axis_name="s")

    @functools.partial(
        pl.kernel,
        out_type=jax.ShapeDtypeStruct((ntok, D), jnp.float32),
        mesh=mesh,
        compiler_params=pltpu.CompilerParams(needs_layout_passes=False),
        scratch_types=[
            pltpu.VMEM((2, K), jnp.int32),
            pltpu.VMEM((2, K, D), jnp.float32),
            pltpu.SemaphoreType.DMA((2,)),
            pltpu.SemaphoreType.DMA((2,)),
        ],
    )
    def sc_gather(ids_hbm, wtab_hbm, out_hbm, idx, rows, semg, semo):
        wid = lax.axis_index("s") * NC + lax.axis_index("c")
        base = wid * tpw

        def issue(c, b):
            pltpu.sync_copy(ids_hbm.at[pl.ds(base + c * K, K)], idx.at[b])
            return pltpu.async_copy(wtab_hbm.at[idx.at[b]], rows.at[b],
                                    semg.at[b])

        gat = {0: issue(0, 0)}
        out = {}
        for c in range(nch):
            b = c & 1
            if c + 1 < nch:
                if c >= 1:
                    out[c - 1].wait()
                gat[c + 1] = issue(c + 1, 1 - b)
            gat.pop(c).wait()
            out[c] = pltpu.async_copy(
                rows.at[b], out_hbm.at[pl.ds(base + c * K, K)], semo.at[b])
        out[nch - 2].wait()
        out[nch - 1].wait()

    return sc_gather


def _tc_ln_body(wsum_ref, pos_ref, tt_ref, tid_ref, g_ref, b_ref, out_ref):
    tidf = tid_ref[...]
    t0 = tt_ref[0:1, :]
    dt = tt_ref[1:2, :] - t0
    y = wsum_ref[...] + pos_ref[...] + (t0 + tidf * dt)
    mean = jnp.mean(y, axis=-1, keepdims=True)
    var = jnp.mean(y * y, axis=-1, keepdims=True) - mean * mean
    inv = lax.rsqrt(var + EPS)
    out_ref[...] = (y - mean) * inv * g_ref[...] + b_ref[...]


def _make_tc_ln(ntok):
    spb = SEQ // TB
    nb = ntok // SEQ
    tok = lambda j, i: (i * spb + j, 0)
    return pl.pallas_call(
        _tc_ln_body,
        grid=(spb, nb),
        in_specs=[
            pl.BlockSpec((TB, D), tok),
            pl.BlockSpec((TB, D), lambda j, i: (j, 0)),
            pl.BlockSpec((2, D), lambda j, i: (0, 0)),
            pl.BlockSpec((TB, 1), tok),
            pl.BlockSpec((1, D), lambda j, i: (0, 0)),
            pl.BlockSpec((1, D), lambda j, i: (0, 0)),
        ],
        out_specs=pl.BlockSpec((TB, D), tok),
        out_shape=jax.ShapeDtypeStruct((ntok, D), jnp.float32),
    )


def kernel(input_ids, token_type_ids, word_emb, pos_emb, tt_emb, gamma, beta):
    b, seq = input_ids.shape
    ntok = b * seq
    ids_flat = input_ids.reshape(ntok).astype(jnp.int32)
    ttf = token_type_ids.reshape(ntok, 1).astype(jnp.float32)
    wsum = _make_sc_gather(ntok)(ids_flat, word_emb)
    out = _make_tc_ln(ntok)(wsum, pos_emb, tt_emb, ttf,
                            gamma.reshape(1, D), beta.reshape(1, D))
    return out.reshape(b, seq, D)


# speedup vs baseline: 5.4874x; 1.1173x over previous
import functools

import jax
import jax.numpy as jnp
from jax import lax
from jax.experimental import pallas as pl
from jax.experimental.pallas import tpu as pltpu
from jax.experimental.pallas import tpu_sc as plsc

D = 1024
SEQ = 2048
NC = 2
NS = 16
NW = NC * NS
K = 32
TB = 512
EPS = 1e-12


def _make_sc_gather(ntok):
    tpw = ntok // NW
    nch = tpw // K
    mesh = plsc.VectorSubcoreMesh(core_axis_name="c", subcore_axis_name="s")

    @functools.partial(
        pl.kernel,
        out_type=jax.ShapeDtypeStruct((ntok, D), jnp.float32),
        mesh=mesh,
        compiler_params=pltpu.CompilerParams(needs_layout_passes=False),
        scratch_types=[
            pltpu.VMEM((2, K), jnp.int32),
            pltpu.VMEM((2, K, D), jnp.float32),
            pltpu.SemaphoreType.DMA((2,)),
            pltpu.SemaphoreType.DMA((2,)),
        ],
    )
    def sc_gather(ids_hbm, wtab_hbm, out_hbm, idx, rows, semg, semo):
        wid = lax.axis_index("s") * NC + lax.axis_index("c")
        base = wid * tpw

        def issue(c, b):
            pltpu.sync_copy(ids_hbm.at[pl.ds(base + c * K, K)], idx.at[b])
            return pltpu.async_copy(wtab_hbm.at[idx.at[b]], rows.at[b],
                                    semg.at[b])

        gat = {0: issue(0, 0)}
        out = {}
        for c in range(nch):
            b = c & 1
            if c + 1 < nch:
                if c >= 1:
                    out[c - 1].wait()
                gat[c + 1] = issue(c + 1, 1 - b)
            gat.pop(c).wait()
            out[c] = pltpu.async_copy(
                rows.at[b], out_hbm.at[pl.ds(base + c * K, K)], semo.at[b])
        out[nch - 2].wait()
        out[nch - 1].wait()

    return sc_gather


def _tc_ln_body(wsum_ref, pos_ref, tt_ref, tid_ref, g_ref, b_ref, out_ref):
    tidf = tid_ref[...]
    t0 = tt_ref[0:1, :]
    dt = tt_ref[1:2, :] - t0
    y = wsum_ref[...] + pos_ref[...] + (t0 + tidf * dt)
    mean = jnp.mean(y, axis=-1, keepdims=True)
    var = jnp.mean(y * y, axis=-1, keepdims=True) - mean * mean
    inv = lax.rsqrt(var + EPS)
    out_ref[...] = (y - mean) * inv * g_ref[...] + b_ref[...]


def _make_tc_ln(ntok):
    spb = SEQ // TB
    nb = ntok // SEQ
    tok = lambda j, i: (i * spb + j, 0)
    return pl.pallas_call(
        _tc_ln_body,
        grid=(spb, nb),
        in_specs=[
            pl.BlockSpec((TB, D), tok),
            pl.BlockSpec((TB, D), lambda j, i: (j, 0)),
            pl.BlockSpec((2, D), lambda j, i: (0, 0)),
            pl.BlockSpec((TB, 1), tok),
            pl.BlockSpec((1, D), lambda j, i: (0, 0)),
            pl.BlockSpec((1, D), lambda j, i: (0, 0)),
        ],
        out_specs=pl.BlockSpec((TB, D), tok),
        out_shape=jax.ShapeDtypeStruct((ntok, D), jnp.float32),
    )


def kernel(input_ids, token_type_ids, word_emb, pos_emb, tt_emb, gamma, beta):
    b, seq = input_ids.shape
    ntok = b * seq
    ids_flat = input_ids.reshape(ntok).astype(jnp.int32)
    ttf = token_type_ids.reshape(ntok, 1).astype(jnp.float32)
    wsum = _make_sc_gather(ntok)(ids_flat, word_emb)
    out = _make_tc_ln(ntok)(wsum, pos_emb, tt_emb, ttf,
                            gamma.reshape(1, D), beta.reshape(1, D))
    return out.reshape(b, seq, D)


# speedup vs baseline: 5.8492x; 1.0659x over previous
import functools

import jax
import jax.numpy as jnp
from jax import lax
from jax.experimental import pallas as pl
from jax.experimental.pallas import tpu as pltpu
from jax.experimental.pallas import tpu_sc as plsc

D = 1024
SEQ = 2048
NC = 2
NS = 16
NW = NC * NS
K = 32
TB = 1024
EPS = 1e-12


def _make_sc_gather(ntok):
    tpw = ntok // NW
    nch = tpw // K
    mesh = plsc.VectorSubcoreMesh(core_axis_name="c", subcore_axis_name="s")

    @functools.partial(
        pl.kernel,
        out_type=jax.ShapeDtypeStruct((ntok, D), jnp.float32),
        mesh=mesh,
        compiler_params=pltpu.CompilerParams(needs_layout_passes=False),
        scratch_types=[
            pltpu.VMEM((2, K), jnp.int32),
            pltpu.VMEM((2, K, D), jnp.float32),
            pltpu.SemaphoreType.DMA((2,)),
            pltpu.SemaphoreType.DMA((2,)),
        ],
    )
    def sc_gather(ids_hbm, wtab_hbm, out_hbm, idx, rows, semg, semo):
        wid = lax.axis_index("s") * NC + lax.axis_index("c")
        base = wid * tpw

        def issue(c, b):
            pltpu.sync_copy(ids_hbm.at[pl.ds(base + c * K, K)], idx.at[b])
            return pltpu.async_copy(wtab_hbm.at[idx.at[b]], rows.at[b],
                                    semg.at[b])

        gat = {0: issue(0, 0)}
        out = {}
        for c in range(nch):
            b = c & 1
            if c + 1 < nch:
                if c >= 1:
                    out[c - 1].wait()
                gat[c + 1] = issue(c + 1, 1 - b)
            gat.pop(c).wait()
            out[c] = pltpu.async_copy(
                rows.at[b], out_hbm.at[pl.ds(base + c * K, K)], semo.at[b])
        out[nch - 2].wait()
        out[nch - 1].wait()

    return sc_gather


def _tc_ln_body(wsum_ref, pos_ref, tt_ref, tid_ref, g_ref, b_ref, out_ref):
    tidf = tid_ref[...]
    t0 = tt_ref[0:1, :]
    dt = tt_ref[1:2, :] - t0
    y = wsum_ref[...] + pos_ref[...] + (t0 + tidf * dt)
    mean = jnp.mean(y, axis=-1, keepdims=True)
    var = jnp.mean(y * y, axis=-1, keepdims=True) - mean * mean
    inv = lax.rsqrt(var + EPS)
    out_ref[...] = (y - mean) * inv * g_ref[...] + b_ref[...]


def _make_tc_ln(ntok):
    spb = SEQ // TB
    nb = ntok // SEQ
    tok = lambda j, i: (i * spb + j, 0)
    return pl.pallas_call(
        _tc_ln_body,
        grid=(spb, nb),
        in_specs=[
            pl.BlockSpec((TB, D), tok),
            pl.BlockSpec((TB, D), lambda j, i: (j, 0)),
            pl.BlockSpec((2, D), lambda j, i: (0, 0)),
            pl.BlockSpec((TB, 1), tok),
            pl.BlockSpec((1, D), lambda j, i: (0, 0)),
            pl.BlockSpec((1, D), lambda j, i: (0, 0)),
        ],
        out_specs=pl.BlockSpec((TB, D), tok),
        out_shape=jax.ShapeDtypeStruct((ntok, D), jnp.float32),
    )


def kernel(input_ids, token_type_ids, word_emb, pos_emb, tt_emb, gamma, beta):
    b, seq = input_ids.shape
    ntok = b * seq
    ids_flat = input_ids.reshape(ntok).astype(jnp.int32)
    ttf = token_type_ids.reshape(ntok, 1).astype(jnp.float32)
    wsum = _make_sc_gather(ntok)(ids_flat, word_emb)
    out = _make_tc_ln(ntok)(wsum, pos_emb, tt_emb, ttf,
                            gamma.reshape(1, D), beta.reshape(1, D))
    return out.reshape(b, seq, D)


# speedup vs baseline: 5.8894x; 1.0069x over previous
import functools

import jax
import jax.numpy as jnp
from jax import lax
from jax.experimental import pallas as pl
from jax.experimental.pallas import tpu as pltpu
from jax.experimental.pallas import tpu_sc as plsc

D = 1024
SEQ = 2048
NC = 2
NS = 16
NW = NC * NS
K = 32
TB = 2048
EPS = 1e-12


def _make_sc_gather(ntok):
    tpw = ntok // NW
    nch = tpw // K
    mesh = plsc.VectorSubcoreMesh(core_axis_name="c", subcore_axis_name="s")

    @functools.partial(
        pl.kernel,
        out_type=jax.ShapeDtypeStruct((ntok, D), jnp.float32),
        mesh=mesh,
        compiler_params=pltpu.CompilerParams(needs_layout_passes=False),
        scratch_types=[
            pltpu.VMEM((2, K), jnp.int32),
            pltpu.VMEM((2, K, D), jnp.float32),
            pltpu.SemaphoreType.DMA((2,)),
            pltpu.SemaphoreType.DMA((2,)),
        ],
    )
    def sc_gather(ids_hbm, wtab_hbm, out_hbm, idx, rows, semg, semo):
        wid = lax.axis_index("s") * NC + lax.axis_index("c")
        base = wid * tpw

        def issue(c, b):
            pltpu.sync_copy(ids_hbm.at[pl.ds(base + c * K, K)], idx.at[b])
            return pltpu.async_copy(wtab_hbm.at[idx.at[b]], rows.at[b],
                                    semg.at[b])

        gat = {0: issue(0, 0)}
        out = {}
        for c in range(nch):
            b = c & 1
            if c + 1 < nch:
                if c >= 1:
                    out[c - 1].wait()
                gat[c + 1] = issue(c + 1, 1 - b)
            gat.pop(c).wait()
            out[c] = pltpu.async_copy(
                rows.at[b], out_hbm.at[pl.ds(base + c * K, K)], semo.at[b])
        out[nch - 2].wait()
        out[nch - 1].wait()

    return sc_gather


def _tc_ln_body(wsum_ref, pos_ref, tt_ref, tid_ref, g_ref, b_ref, out_ref):
    tidf = tid_ref[...]
    t0 = tt_ref[0:1, :]
    dt = tt_ref[1:2, :] - t0
    y = wsum_ref[...] + pos_ref[...] + (t0 + tidf * dt)
    mean = jnp.mean(y, axis=-1, keepdims=True)
    var = jnp.mean(y * y, axis=-1, keepdims=True) - mean * mean
    inv = lax.rsqrt(var + EPS)
    out_ref[...] = (y - mean) * inv * g_ref[...] + b_ref[...]


def _make_tc_ln(ntok):
    spb = SEQ // TB
    nb = ntok // SEQ
    tok = lambda j, i: (i * spb + j, 0)
    return pl.pallas_call(
        _tc_ln_body,
        grid=(spb, nb),
        in_specs=[
            pl.BlockSpec((TB, D), tok),
            pl.BlockSpec((TB, D), lambda j, i: (j, 0)),
            pl.BlockSpec((2, D), lambda j, i: (0, 0)),
            pl.BlockSpec((TB, 1), tok),
            pl.BlockSpec((1, D), lambda j, i: (0, 0)),
            pl.BlockSpec((1, D), lambda j, i: (0, 0)),
        ],
        out_specs=pl.BlockSpec((TB, D), tok),
        out_shape=jax.ShapeDtypeStruct((ntok, D), jnp.float32),
    )


def kernel(input_ids, token_type_ids, word_emb, pos_emb, tt_emb, gamma, beta):
    b, seq = input_ids.shape
    ntok = b * seq
    ids_flat = input_ids.reshape(ntok).astype(jnp.int32)
    ttf = token_type_ids.reshape(ntok, 1).astype(jnp.float32)
    wsum = _make_sc_gather(ntok)(ids_flat, word_emb)
    out = _make_tc_ln(ntok)(wsum, pos_emb, tt_emb, ttf,
                            gamma.reshape(1, D), beta.reshape(1, D))
    return out.reshape(b, seq, D)
